# baseline (speedup 1.0000x reference)
import jax
import jax.numpy as jnp
from jax.experimental import pallas as pl
from jax.experimental.pallas import tpu as pltpu

_TILE = 32768


_CHUNK = 512


def _mlp_kernel(xt_ref, w1t_ref, b1c_ref, w2t_ref, o_ref):
    w1t = w1t_ref[...]
    b1c = b1c_ref[...]
    w2t = w2t_ref[...]

    def body(c, _):
        xc = xt_ref[:, pl.ds(c * _CHUNK, _CHUNK)].astype(jnp.bfloat16)
        h = jax.lax.dot_general(
            w1t, xc, (((1,), (0,)), ((), ())),
            preferred_element_type=jnp.float32,
        )
        h = jnp.maximum(h + b1c, 0.0).astype(jnp.bfloat16)
        o_ref[:, pl.ds(c * _CHUNK, _CHUNK)] = jax.lax.dot_general(
            w2t, h, (((1,), (0,)), ((), ())),
            preferred_element_type=jnp.float32,
        )
        return ()

    jax.lax.fori_loop(0, _TILE // _CHUNK, body, ())


def kernel(x, w1_aug, w2_aug):
    x = jnp.asarray(x, jnp.float32)
    B = x.shape[0]
    B_pad = ((B + _TILE - 1) // _TILE) * _TILE

    xt = x.T
    if B_pad != B:
        xt = jnp.pad(xt, ((0, 0), (0, B_pad - B)))

    w1t = w1_aug[:12, :].T.astype(jnp.bfloat16)
    b1c = w1_aug[12:13, :].T
    w2t = jnp.zeros((8, 128), jnp.bfloat16).at[:4, :].set(w2_aug[:, :4].T.astype(jnp.bfloat16))

    ot = pl.pallas_call(
        _mlp_kernel,
        out_shape=jax.ShapeDtypeStruct((8, B_pad), jnp.float32),
        grid=(B_pad // _TILE,),
        in_specs=[
            pl.BlockSpec((12, _TILE), lambda i: (0, i)),
            pl.BlockSpec((128, 12), lambda i: (0, 0)),
            pl.BlockSpec((128, 1), lambda i: (0, 0)),
            pl.BlockSpec((8, 128), lambda i: (0, 0)),
        ],
        out_specs=pl.BlockSpec((8, _TILE), lambda i: (0, i)),
        compiler_params=pltpu.CompilerParams(
            dimension_semantics=("parallel",)
        ),
    )(xt, w1t, b1c, w2t)

    return ot[:4, :B].T


# speedup vs baseline: 4.3219x; 4.3219x over previous
import jax
import jax.numpy as jnp
from jax.experimental import pallas as pl
from jax.experimental.pallas import tpu as pltpu

_TILE = 65536


def _mlp_kernel(xt_ref, w1t_ref, b1c_ref, w2t_ref, o_ref):
    h = jax.lax.dot_general(
        w1t_ref[...], xt_ref[...].astype(jnp.bfloat16), (((1,), (0,)), ((), ())),
        preferred_element_type=jnp.float32,
    )
    h = jnp.maximum(h + b1c_ref[...], 0.0).astype(jnp.bfloat16)
    o_ref[...] = jax.lax.dot_general(
        w2t_ref[...], h, (((1,), (0,)), ((), ())),
        preferred_element_type=jnp.float32,
    )


def kernel(x, w1_aug, w2_aug):
    x = jnp.asarray(x, jnp.float32)
    B = x.shape[0]
    B_pad = ((B + _TILE - 1) // _TILE) * _TILE

    xt = x.T
    if B_pad != B:
        xt = jnp.pad(xt, ((0, 0), (0, B_pad - B)))

    w1t = w1_aug[:12, :].T.astype(jnp.bfloat16)
    b1c = w1_aug[12:13, :].T
    w2t = jnp.zeros((8, 128), jnp.bfloat16).at[:4, :].set(w2_aug[:, :4].T.astype(jnp.bfloat16))

    ot = pl.pallas_call(
        _mlp_kernel,
        out_shape=jax.ShapeDtypeStruct((8, B_pad), jnp.float32),
        grid=(B_pad // _TILE,),
        in_specs=[
            pl.BlockSpec((12, _TILE), lambda i: (0, i)),
            pl.BlockSpec((128, 12), lambda i: (0, 0)),
            pl.BlockSpec((128, 1), lambda i: (0, 0)),
            pl.BlockSpec((8, 128), lambda i: (0, 0)),
        ],
        out_specs=pl.BlockSpec((8, _TILE), lambda i: (0, i)),
        compiler_params=pltpu.CompilerParams(
            dimension_semantics=("parallel",)
        ),
    )(xt, w1t, b1c, w2t)

    return ot[:4, :B].T


# speedup vs baseline: 4.3388x; 1.0039x over previous
import jax
import jax.numpy as jnp
from jax.experimental import pallas as pl
from jax.experimental.pallas import tpu as pltpu

_TILE = 32768


def _mlp_kernel(xt_ref, w1t_ref, b1c_ref, w2t_ref, o_ref):
    h = jax.lax.dot_general(
        w1t_ref[...], xt_ref[...].astype(jnp.bfloat16), (((1,), (0,)), ((), ())),
        preferred_element_type=jnp.float32,
    )
    h = jnp.maximum(h + b1c_ref[...], 0.0).astype(jnp.bfloat16)
    o_ref[...] = jax.lax.dot_general(
        w2t_ref[...], h, (((1,), (0,)), ((), ())),
        preferred_element_type=jnp.float32,
    )


def kernel(x, w1_aug, w2_aug):
    x = jnp.asarray(x, jnp.float32)
    B = x.shape[0]
    B_pad = ((B + _TILE - 1) // _TILE) * _TILE

    xt = x.T
    if B_pad != B:
        xt = jnp.pad(xt, ((0, 0), (0, B_pad - B)))

    w1t = w1_aug[:12, :104].T.astype(jnp.bfloat16)
    b1c = w1_aug[12:13, :104].T.at[100, 0].set(1.0)
    w2t = (jnp.zeros((8, 104), jnp.float32)
           .at[:4, :100].set(w2_aug[:100, :4].T)
           .at[:4, 100].set(w2_aug[127, :4])).astype(jnp.bfloat16)

    ot = pl.pallas_call(
        _mlp_kernel,
        out_shape=jax.ShapeDtypeStruct((8, B_pad), jnp.float32),
        grid=(B_pad // _TILE,),
        in_specs=[
            pl.BlockSpec((12, _TILE), lambda i: (0, i)),
            pl.BlockSpec((104, 12), lambda i: (0, 0)),
            pl.BlockSpec((104, 1), lambda i: (0, 0)),
            pl.BlockSpec((8, 104), lambda i: (0, 0)),
        ],
        out_specs=pl.BlockSpec((8, _TILE), lambda i: (0, i)),
        compiler_params=pltpu.CompilerParams(
            dimension_semantics=("parallel",)
        ),
    )(xt, w1t, b1c, w2t)

    return ot[:4, :B].T


# speedup vs baseline: 4.9989x; 1.1521x over previous
import jax
import jax.numpy as jnp
from jax.experimental import pallas as pl
from jax.experimental.pallas import tpu as pltpu

_TILE = 32768


def _mlp_kernel(xt_ref, w1t_ref, b1c_ref, w2t_ref, o_ref):
    h = jax.lax.dot_general(
        w1t_ref[...], xt_ref[...].astype(jnp.bfloat16), (((1,), (0,)), ((), ())),
        preferred_element_type=jnp.float32,
    )
    h = jnp.maximum(h + b1c_ref[...], 0.0).astype(jnp.bfloat16)
    o_ref[...] = jax.lax.dot_general(
        w2t_ref[...], h, (((1,), (0,)), ((), ())),
        preferred_element_type=jnp.float32,
    )[:4, :]


def kernel(x, w1_aug, w2_aug):
    x = jnp.asarray(x, jnp.float32)
    B = x.shape[0]
    B_pad = ((B + _TILE - 1) // _TILE) * _TILE

    xt = x.T
    if B_pad != B:
        xt = jnp.pad(xt, ((0, 0), (0, B_pad - B)))

    w1t = w1_aug[:12, :104].T.astype(jnp.bfloat16)
    b1c = w1_aug[12:13, :104].T.at[100, 0].set(1.0)
    w2t = (jnp.zeros((8, 104), jnp.float32)
           .at[:4, :100].set(w2_aug[:100, :4].T)
           .at[:4, 100].set(w2_aug[127, :4])).astype(jnp.bfloat16)

    ot = pl.pallas_call(
        _mlp_kernel,
        out_shape=jax.ShapeDtypeStruct((4, B_pad), jnp.float32),
        grid=(B_pad // _TILE,),
        in_specs=[
            pl.BlockSpec((12, _TILE), lambda i: (0, i)),
            pl.BlockSpec((104, 12), lambda i: (0, 0)),
            pl.BlockSpec((104, 1), lambda i: (0, 0)),
            pl.BlockSpec((8, 104), lambda i: (0, 0)),
        ],
        out_specs=pl.BlockSpec((4, _TILE), lambda i: (0, i)),
        compiler_params=pltpu.CompilerParams(
            dimension_semantics=("parallel",)
        ),
    )(xt, w1t, b1c, w2t)

    return ot[:, :B].T


# speedup vs baseline: 5.0156x; 1.0033x over previous
import jax
import jax.numpy as jnp
from jax.experimental import pallas as pl
from jax.experimental.pallas import tpu as pltpu

_TILE = 32768


def _mlp_kernel(xt_ref, w1t_ref, b1c_ref, w2t_ref, o_ref):
    w1t = w1t_ref[...]
    b1c = b1c_ref[...]
    w2t = w2t_ref[...]
    half = _TILE // 2
    for c in range(2):
        xc = xt_ref[:, c * half:(c + 1) * half].astype(jnp.bfloat16)
        h = jax.lax.dot_general(
            w1t, xc, (((1,), (0,)), ((), ())),
            preferred_element_type=jnp.float32,
        )
        h = jnp.maximum(h + b1c, 0.0).astype(jnp.bfloat16)
        o_ref[:, c * half:(c + 1) * half] = jax.lax.dot_general(
            w2t, h, (((1,), (0,)), ((), ())),
            preferred_element_type=jnp.float32,
        )[:4, :]


def kernel(x, w1_aug, w2_aug):
    x = jnp.asarray(x, jnp.float32)
    B = x.shape[0]
    B_pad = ((B + _TILE - 1) // _TILE) * _TILE

    xt = x.T
    if B_pad != B:
        xt = jnp.pad(xt, ((0, 0), (0, B_pad - B)))

    w1t = w1_aug[:12, :104].T.astype(jnp.bfloat16)
    b1c = w1_aug[12:13, :104].T.at[100, 0].set(1.0)
    w2t = (jnp.zeros((8, 104), jnp.float32)
           .at[:4, :100].set(w2_aug[:100, :4].T)
           .at[:4, 100].set(w2_aug[127, :4])).astype(jnp.bfloat16)

    ot = pl.pallas_call(
        _mlp_kernel,
        out_shape=jax.ShapeDtypeStruct((4, B_pad), jnp.float32),
        grid=(B_pad // _TILE,),
        in_specs=[
            pl.BlockSpec((12, _TILE), lambda i: (0, i)),
            pl.BlockSpec((104, 12), lambda i: (0, 0)),
            pl.BlockSpec((104, 1), lambda i: (0, 0)),
            pl.BlockSpec((8, 104), lambda i: (0, 0)),
        ],
        out_specs=pl.BlockSpec((4, _TILE), lambda i: (0, i)),
        compiler_params=pltpu.CompilerParams(
            dimension_semantics=("parallel",)
        ),
    )(xt, w1t, b1c, w2t)

    return ot[:, :B].T
